# baseline (speedup 1.0000x reference)
import functools

import jax
import jax.numpy as jnp
from jax import lax
from jax.experimental import pallas as pl
from jax.experimental.pallas import tpu as pltpu
from jax.experimental.pallas import tpu_sc as plsc

N = 10000
D = 128
E = 160000
R = 2
L = 16
NS = 16
N_PAD = 10240
ROWS_PER_SUB = N_PAD // NS
E_PER_SUB = E // NS
CHUNK = 96
NCHUNK = 105
E_SUB_PAD = NCHUNK * CHUNK
FAKE = E_SUB_PAD - E_PER_SUB


def _sc_segment_sum(x, src_flat, dst_flat):
    mesh = plsc.VectorSubcoreMesh(core_axis_name="c", subcore_axis_name="s")

    @functools.partial(
        pl.kernel,
        out_type=(jax.ShapeDtypeStruct((R, N_PAD, D), jnp.float32),
                  jax.ShapeDtypeStruct((R, N_PAD), jnp.float32)),
        mesh=mesh,
        scratch_types=[
            pltpu.VMEM((CHUNK,), jnp.int32),
            pltpu.VMEM((CHUNK,), jnp.int32),
            pltpu.VMEM((CHUNK,), jnp.int32),
            pltpu.VMEM((CHUNK,), jnp.int32),
            pltpu.VMEM((CHUNK,), jnp.int32),
            pltpu.VMEM((CHUNK,), jnp.int32),
            pltpu.VMEM((CHUNK, D), jnp.float32),
            pltpu.VMEM((CHUNK, D), jnp.float32),
            pltpu.VMEM((CHUNK, D), jnp.float32),
            pltpu.VMEM((CHUNK,), jnp.float32),
            pltpu.VMEM((CHUNK,), jnp.float32),
            pltpu.VMEM_SHARED((N_PAD, D), jnp.float32),
            pltpu.VMEM_SHARED((N_PAD,), jnp.float32),
            pltpu.SemaphoreType.DMA,
            pltpu.SemaphoreType.DMA,
            pltpu.SemaphoreType.DMA,
            pltpu.SemaphoreType.DMA,
            pltpu.SemaphoreType.DMA,
            pltpu.SemaphoreType.DMA,
            pltpu.SemaphoreType.DMA,
            pltpu.SemaphoreType.DMA,
            pltpu.SemaphoreType.DMA,
        ],
    )
    def sc_kernel(x_hbm, src_hbm, dst_hbm, out_hbm, deg_hbm,
                  srcb0, srcb1, srcb2, dstb0, dstb1, dstb2,
                  rows0, rows1, rows2, ones_v, zeros_v, acc_sh, deg_sh,
                  ss0, ss1, ss2, sd0, sd1, sd2, sg0, sg1, sg2):
        c = lax.axis_index("c")
        s = lax.axis_index("s")
        ebase = (c * NS + s) * E_SUB_PAD

        srcb = (srcb0, srcb1, srcb2)
        dstb = (dstb0, dstb1, dstb2)
        rows = (rows0, rows1, rows2)
        ss = (ss0, ss1, ss2)
        sd = (sd0, sd1, sd2)
        sg = (sg0, sg1, sg2)

        one = jnp.full((L,), 1.0, jnp.float32)
        zero = jnp.zeros((L,), jnp.float32)

        @pl.loop(0, CHUNK, step=L)
        def _(i):
            ones_v[pl.ds(i, L)] = one
            zeros_v[pl.ds(i, L)] = zero

        @pl.loop(0, 64)
        def _(i):
            @pl.loop(0, D, step=L)
            def _(j):
                rows0[i, pl.ds(j, L)] = zero

        row0 = s * ROWS_PER_SUB

        @pl.loop(0, ROWS_PER_SUB, step=64)
        def _(r0):
            pltpu.sync_copy(rows0.at[pl.ds(0, 64)],
                            acc_sh.at[pl.ds(row0 + r0, 64)])
            pltpu.sync_copy(zeros_v.at[pl.ds(0, 64)],
                            deg_sh.at[pl.ds(row0 + r0, 64)])

        plsc.subcore_barrier()

        def idx_start(k, b):
            pltpu.make_async_copy(
                src_hbm.at[pl.ds(ebase + k * CHUNK, CHUNK)],
                srcb[b], ss[b]).start()
            pltpu.make_async_copy(
                dst_hbm.at[pl.ds(ebase + k * CHUNK, CHUNK)],
                dstb[b], sd[b]).start()

        def idx_wait(b):
            pltpu.make_async_copy(
                src_hbm.at[pl.ds(ebase, CHUNK)], srcb[b], ss[b]).wait()

        def gather_cp(b):
            return pltpu.make_async_copy(x_hbm.at[srcb[b]], rows[b], sg[b])

        def turn(k, b):
            b2 = (b + 2) % 3
            gather_cp(b).wait()
            pltpu.make_async_copy(
                dst_hbm.at[pl.ds(ebase, CHUNK)], dstb[b], sd[b]).wait()
            pltpu.sync_copy(rows[b], acc_sh.at[dstb[b]], add=True)
            pltpu.sync_copy(ones_v, deg_sh.at[dstb[b]], add=True)

            @pl.when(k + 3 < NCHUNK)
            def _():
                idx_start(k + 3, b)

            @pl.when(k + 2 < NCHUNK)
            def _():
                idx_wait(b2)
                gather_cp(b2).start()

        idx_start(0, 0)
        idx_start(1, 1)
        idx_start(2, 2)
        idx_wait(0)
        gather_cp(0).start()
        idx_wait(1)
        gather_cp(1).start()

        @pl.loop(0, NCHUNK, step=3)
        def _(a):
            turn(a, 0)
            turn(a + 1, 1)
            turn(a + 2, 2)

        plsc.subcore_barrier()

        pltpu.sync_copy(acc_sh.at[pl.ds(row0, ROWS_PER_SUB)],
                        out_hbm.at[c, pl.ds(row0, ROWS_PER_SUB)])
        pltpu.sync_copy(deg_sh.at[pl.ds(row0, ROWS_PER_SUB)],
                        deg_hbm.at[c, pl.ds(row0, ROWS_PER_SUB)])

    return sc_kernel(x, src_flat, dst_flat)


_BR = 1000


def _tc_body(s_ref, deg_ref, x_ref, w0_ref, w1_ref, lw_ref, b_ref, o_ref):
    dn = (((1,), (0,)), ((), ()))
    hp = lax.Precision.HIGHEST
    d0 = jnp.maximum(deg_ref[0], 1.0)
    d1 = jnp.maximum(deg_ref[1], 1.0)
    acc = lax.dot_general(s_ref[0], w0_ref[...], dn, precision=hp) / d0
    acc = acc + lax.dot_general(s_ref[1], w1_ref[...], dn, precision=hp) / d1
    acc = acc + lax.dot_general(x_ref[...], lw_ref[...], dn, precision=hp)
    o_ref[...] = jnp.tanh(acc + b_ref[...])


def _tc_combine(S, deg3, x, W0, W1, loop_w, h_bias_row):
    return pl.pallas_call(
        _tc_body,
        grid=(N // _BR,),
        in_specs=[
            pl.BlockSpec((R, _BR, D), lambda i: (0, i, 0)),
            pl.BlockSpec((R, _BR, 1), lambda i: (0, i, 0)),
            pl.BlockSpec((_BR, D), lambda i: (i, 0)),
            pl.BlockSpec((D, D), lambda i: (0, 0)),
            pl.BlockSpec((D, D), lambda i: (0, 0)),
            pl.BlockSpec((D, D), lambda i: (0, 0)),
            pl.BlockSpec((1, D), lambda i: (0, 0)),
        ],
        out_specs=pl.BlockSpec((_BR, D), lambda i: (i, 0)),
        out_shape=jax.ShapeDtypeStruct((N, D), jnp.float32),
    )(S, deg3, x, W0, W1, loop_w, h_bias_row)


def _pad_edges(idx, fake):
    return jnp.concatenate(
        [idx.reshape(NS, E_PER_SUB), fake], axis=1).reshape(-1)


def kernel(x, W, loop_weight, h_bias, edge_index_rel0, edge_index_rel1):
    fake_src = jnp.broadcast_to(
        (jnp.arange(FAKE, dtype=jnp.int32) * 125) % N, (NS, FAKE))
    fake_dst = jnp.broadcast_to(
        N + (jnp.arange(FAKE, dtype=jnp.int32) * 3) % (N_PAD - N),
        (NS, FAKE))
    src_flat = jnp.concatenate(
        [_pad_edges(edge_index_rel0[0], fake_src),
         _pad_edges(edge_index_rel1[0], fake_src)])
    dst_flat = jnp.concatenate(
        [_pad_edges(edge_index_rel0[1], fake_dst),
         _pad_edges(edge_index_rel1[1], fake_dst)])
    S, deg = _sc_segment_sum(x, src_flat, dst_flat)
    deg3 = deg.reshape(R, N_PAD, 1)
    return _tc_combine(S, deg3, x, W[0], W[1], loop_weight,
                       h_bias.reshape(1, D))


# speedup vs baseline: 1.1030x; 1.1030x over previous
import functools

import jax
import jax.numpy as jnp
from jax import lax
from jax.experimental import pallas as pl
from jax.experimental.pallas import tpu as pltpu
from jax.experimental.pallas import tpu_sc as plsc

N = 10000
D = 128
E = 160000
R = 2
L = 16
NS = 16
N_PAD = 10240
ROWS_PER_SUB = N_PAD // NS
E_PER_SUB = E // NS
CHUNK = 80
NCHUNK = 126
E_SUB_PAD = NCHUNK * CHUNK
FAKE = E_SUB_PAD - E_PER_SUB


def _sc_segment_sum(x, src_flat, dst_flat):
    mesh = plsc.VectorSubcoreMesh(core_axis_name="c", subcore_axis_name="s")

    @functools.partial(
        pl.kernel,
        out_type=(jax.ShapeDtypeStruct((R, N_PAD, D), jnp.float32),
                  jax.ShapeDtypeStruct((R, N_PAD), jnp.float32)),
        mesh=mesh,
        scratch_types=[
            pltpu.VMEM((E_SUB_PAD,), jnp.int32),
            pltpu.VMEM((CHUNK,), jnp.int32),
            pltpu.VMEM((CHUNK,), jnp.int32),
            pltpu.VMEM((CHUNK,), jnp.int32),
            pltpu.VMEM((CHUNK, D), jnp.float32),
            pltpu.VMEM((CHUNK, D), jnp.float32),
            pltpu.VMEM((CHUNK, D), jnp.float32),
            pltpu.VMEM((CHUNK,), jnp.float32),
            pltpu.VMEM((CHUNK,), jnp.float32),
            pltpu.VMEM_SHARED((N_PAD, D), jnp.float32),
            pltpu.VMEM_SHARED((N_PAD,), jnp.float32),
            pltpu.SemaphoreType.DMA,
            pltpu.SemaphoreType.DMA,
            pltpu.SemaphoreType.DMA,
            pltpu.SemaphoreType.DMA,
            pltpu.SemaphoreType.DMA,
            pltpu.SemaphoreType.DMA,
        ],
    )
    def sc_kernel(x_hbm, src_hbm, dst_hbm, out_hbm, deg_hbm,
                  src_v, dst0, dst1, dst2, rows0, rows1, rows2,
                  ones_v, zeros_v, acc_sh, deg_sh,
                  sg0, sg1, sg2, sd0, sd1, sd2):
        c = lax.axis_index("c")
        s = lax.axis_index("s")
        ebase = (c * NS + s) * E_SUB_PAD

        one = jnp.full((L,), 1.0, jnp.float32)
        zero = jnp.zeros((L,), jnp.float32)

        @pl.loop(0, CHUNK, step=L)
        def _(i):
            ones_v[pl.ds(i, L)] = one
            zeros_v[pl.ds(i, L)] = zero

        @pl.loop(0, 64)
        def _(i):
            @pl.loop(0, D, step=L)
            def _(j):
                rows0[i, pl.ds(j, L)] = zero

        row0 = s * ROWS_PER_SUB

        @pl.loop(0, ROWS_PER_SUB, step=64)
        def _(r0):
            pltpu.sync_copy(rows0.at[pl.ds(0, 64)],
                            acc_sh.at[pl.ds(row0 + r0, 64)])
            pltpu.sync_copy(zeros_v.at[pl.ds(0, 64)],
                            deg_sh.at[pl.ds(row0 + r0, 64)])

        pltpu.async_copy(src_hbm.at[pl.ds(ebase, E_SUB_PAD)], src_v, sg0).wait()

        plsc.subcore_barrier()

        def gather(k, buf, sem):
            return pltpu.make_async_copy(
                x_hbm.at[src_v.at[pl.ds(k * CHUNK, CHUNK)]], buf, sem)

        def dstcp(k, buf, sem):
            return pltpu.make_async_copy(
                dst_hbm.at[pl.ds(ebase + k * CHUNK, CHUNK)], buf, sem)

        def consume(k, buf, dbuf, sg, sd):
            gather(k, buf, sg).wait()
            dstcp(k, dbuf, sd).wait()
            pltpu.sync_copy(buf, acc_sh.at[dbuf], add=True)
            pltpu.sync_copy(ones_v, deg_sh.at[dbuf], add=True)

        def prefetch(k, buf, dbuf, sg, sd):
            @pl.when(k < NCHUNK)
            def _():
                dstcp(k, dbuf, sd).start()
                gather(k, buf, sg).start()

        dstcp(0, dst0, sd0).start()
        gather(0, rows0, sg0).start()
        dstcp(1, dst1, sd1).start()
        gather(1, rows1, sg1).start()

        @pl.loop(0, NCHUNK, step=3)
        def _(a):
            prefetch(a + 2, rows2, dst2, sg2, sd2)
            consume(a, rows0, dst0, sg0, sd0)
            prefetch(a + 3, rows0, dst0, sg0, sd0)
            consume(a + 1, rows1, dst1, sg1, sd1)
            prefetch(a + 4, rows1, dst1, sg1, sd1)
            consume(a + 2, rows2, dst2, sg2, sd2)

        plsc.subcore_barrier()

        pltpu.sync_copy(acc_sh.at[pl.ds(row0, ROWS_PER_SUB)],
                        out_hbm.at[c, pl.ds(row0, ROWS_PER_SUB)])
        pltpu.sync_copy(deg_sh.at[pl.ds(row0, ROWS_PER_SUB)],
                        deg_hbm.at[c, pl.ds(row0, ROWS_PER_SUB)])

    return sc_kernel(x, src_flat, dst_flat)


_BR = 1000


def _tc_body(s_ref, deg_ref, x_ref, w0_ref, w1_ref, lw_ref, b_ref, o_ref):
    dn = (((1,), (0,)), ((), ()))
    hp = lax.Precision.HIGHEST
    d0 = jnp.maximum(deg_ref[0], 1.0)
    d1 = jnp.maximum(deg_ref[1], 1.0)
    acc = lax.dot_general(s_ref[0], w0_ref[...], dn, precision=hp) / d0
    acc = acc + lax.dot_general(s_ref[1], w1_ref[...], dn, precision=hp) / d1
    acc = acc + lax.dot_general(x_ref[...], lw_ref[...], dn, precision=hp)
    o_ref[...] = jnp.tanh(acc + b_ref[...])


def _tc_combine(S, deg3, x, W0, W1, loop_w, h_bias_row):
    return pl.pallas_call(
        _tc_body,
        grid=(N // _BR,),
        in_specs=[
            pl.BlockSpec((R, _BR, D), lambda i: (0, i, 0)),
            pl.BlockSpec((R, _BR, 1), lambda i: (0, i, 0)),
            pl.BlockSpec((_BR, D), lambda i: (i, 0)),
            pl.BlockSpec((D, D), lambda i: (0, 0)),
            pl.BlockSpec((D, D), lambda i: (0, 0)),
            pl.BlockSpec((D, D), lambda i: (0, 0)),
            pl.BlockSpec((1, D), lambda i: (0, 0)),
        ],
        out_specs=pl.BlockSpec((_BR, D), lambda i: (i, 0)),
        out_shape=jax.ShapeDtypeStruct((N, D), jnp.float32),
    )(S, deg3, x, W0, W1, loop_w, h_bias_row)


def _pad_edges(idx, fake):
    return jnp.concatenate(
        [idx.reshape(NS, E_PER_SUB), fake], axis=1).reshape(-1)


def kernel(x, W, loop_weight, h_bias, edge_index_rel0, edge_index_rel1):
    fake_src = jnp.broadcast_to(
        (jnp.arange(FAKE, dtype=jnp.int32) * 125) % N, (NS, FAKE))
    fake_dst = jnp.broadcast_to(
        N + (jnp.arange(FAKE, dtype=jnp.int32) * 3) % (N_PAD - N),
        (NS, FAKE))
    src_flat = jnp.concatenate(
        [_pad_edges(edge_index_rel0[0], fake_src),
         _pad_edges(edge_index_rel1[0], fake_src)])
    dst_flat = jnp.concatenate(
        [_pad_edges(edge_index_rel0[1], fake_dst),
         _pad_edges(edge_index_rel1[1], fake_dst)])
    S, deg = _sc_segment_sum(x, src_flat, dst_flat)
    deg3 = deg.reshape(R, N_PAD, 1)
    return _tc_combine(S, deg3, x, W[0], W[1], loop_weight,
                       h_bias.reshape(1, D))


# speedup vs baseline: 1.1958x; 1.0841x over previous
import functools

import jax
import jax.numpy as jnp
from jax import lax
from jax.experimental import pallas as pl
from jax.experimental.pallas import tpu as pltpu
from jax.experimental.pallas import tpu_sc as plsc

N = 10000
D = 128
E = 160000
R = 2
L = 16
NS = 16
N_PAD = 10240
ROWS_PER_SUB = N_PAD // NS
E_PER_SUB = E // NS
CHUNK = 80
NCHUNK = E_PER_SUB // CHUNK
NMAIN = ((NCHUNK - 2) // 3) * 3


def _sc_segment_sum(x, src_flat, dst_flat):
    mesh = plsc.VectorSubcoreMesh(core_axis_name="c", subcore_axis_name="s")

    @functools.partial(
        pl.kernel,
        out_type=(jax.ShapeDtypeStruct((R, N_PAD, D), jnp.float32),
                  jax.ShapeDtypeStruct((R, N_PAD), jnp.float32)),
        mesh=mesh,
        scratch_types=[
            pltpu.VMEM((E_PER_SUB,), jnp.int32),
            pltpu.VMEM((CHUNK,), jnp.int32),
            pltpu.VMEM((CHUNK,), jnp.int32),
            pltpu.VMEM((CHUNK,), jnp.int32),
            pltpu.VMEM((CHUNK, D), jnp.float32),
            pltpu.VMEM((CHUNK, D), jnp.float32),
            pltpu.VMEM((CHUNK, D), jnp.float32),
            pltpu.VMEM((CHUNK,), jnp.float32),
            pltpu.VMEM((CHUNK,), jnp.float32),
            pltpu.VMEM_SHARED((N_PAD, D), jnp.float32),
            pltpu.VMEM_SHARED((N_PAD,), jnp.float32),
            pltpu.SemaphoreType.DMA,
            pltpu.SemaphoreType.DMA,
            pltpu.SemaphoreType.DMA,
            pltpu.SemaphoreType.DMA,
            pltpu.SemaphoreType.DMA,
            pltpu.SemaphoreType.DMA,
        ],
    )
    def sc_kernel(x_hbm, src_hbm, dst_hbm, out_hbm, deg_hbm,
                  src_v, dst0, dst1, dst2, rows0, rows1, rows2,
                  ones_v, zeros_v, acc_sh, deg_sh,
                  sg0, sg1, sg2, sd0, sd1, sd2):
        c = lax.axis_index("c")
        s = lax.axis_index("s")
        ebase = (c * NS + s) * E_PER_SUB

        one = jnp.full((L,), 1.0, jnp.float32)
        zero = jnp.zeros((L,), jnp.float32)

        @pl.loop(0, CHUNK, step=L)
        def _(i):
            ones_v[pl.ds(i, L)] = one
            zeros_v[pl.ds(i, L)] = zero

        @pl.loop(0, 64)
        def _(i):
            @pl.loop(0, D, step=L)
            def _(j):
                rows0[i, pl.ds(j, L)] = zero

        row0 = s * ROWS_PER_SUB

        @pl.loop(0, ROWS_PER_SUB, step=64)
        def _(r0):
            pltpu.sync_copy(rows0.at[pl.ds(0, 64)],
                            acc_sh.at[pl.ds(row0 + r0, 64)])
            pltpu.sync_copy(zeros_v.at[pl.ds(0, 64)],
                            deg_sh.at[pl.ds(row0 + r0, 64)])

        pltpu.async_copy(src_hbm.at[pl.ds(ebase, E_PER_SUB)], src_v, sg0).wait()

        plsc.subcore_barrier()

        def gather(k, buf, sem):
            return pltpu.make_async_copy(
                x_hbm.at[src_v.at[pl.ds(k * CHUNK, CHUNK)]], buf, sem)

        def dstcp(k, buf, sem):
            return pltpu.make_async_copy(
                dst_hbm.at[pl.ds(ebase + k * CHUNK, CHUNK)], buf, sem)

        def consume(k, buf, dbuf, sg, sd):
            gather(k, buf, sg).wait()
            dstcp(k, dbuf, sd).wait()
            pltpu.sync_copy(buf, acc_sh.at[dbuf], add=True)
            pltpu.sync_copy(ones_v, deg_sh.at[dbuf], add=True)

        def prefetch(k, buf, dbuf, sg, sd):
            @pl.when(k < NCHUNK)
            def _():
                dstcp(k, dbuf, sd).start()
                gather(k, buf, sg).start()

        dstcp(0, dst0, sd0).start()
        gather(0, rows0, sg0).start()
        dstcp(1, dst1, sd1).start()
        gather(1, rows1, sg1).start()

        @pl.loop(0, NMAIN, step=3)
        def _(a):
            prefetch(a + 2, rows2, dst2, sg2, sd2)
            consume(a, rows0, dst0, sg0, sd0)
            prefetch(a + 3, rows0, dst0, sg0, sd0)
            consume(a + 1, rows1, dst1, sg1, sd1)
            prefetch(a + 4, rows1, dst1, sg1, sd1)
            consume(a + 2, rows2, dst2, sg2, sd2)

        consume(NMAIN, rows0, dst0, sg0, sd0)
        consume(NMAIN + 1, rows1, dst1, sg1, sd1)

        plsc.subcore_barrier()

        pltpu.sync_copy(acc_sh.at[pl.ds(row0, ROWS_PER_SUB)],
                        out_hbm.at[c, pl.ds(row0, ROWS_PER_SUB)])
        pltpu.sync_copy(deg_sh.at[pl.ds(row0, ROWS_PER_SUB)],
                        deg_hbm.at[c, pl.ds(row0, ROWS_PER_SUB)])

    return sc_kernel(x, src_flat, dst_flat)


_BR = 1000
_DN = (((1,), (0,)), ((), ()))
_HP = lax.Precision.HIGHEST


def _tc_pre_body(x_ref, lw_ref, b_ref, o_ref):
    o_ref[...] = lax.dot_general(
        x_ref[...], lw_ref[...], _DN, precision=_HP) + b_ref[...]


def _tc_pre(x, loop_w, h_bias_row):
    return pl.pallas_call(
        _tc_pre_body,
        grid=(N // _BR,),
        in_specs=[
            pl.BlockSpec((_BR, D), lambda i: (i, 0)),
            pl.BlockSpec((D, D), lambda i: (0, 0)),
            pl.BlockSpec((1, D), lambda i: (0, 0)),
        ],
        out_specs=pl.BlockSpec((_BR, D), lambda i: (i, 0)),
        out_shape=jax.ShapeDtypeStruct((N, D), jnp.float32),
    )(x, loop_w, h_bias_row)


def _tc_post_body(s_ref, deg_ref, p_ref, w0_ref, w1_ref, o_ref):
    d0 = jnp.maximum(deg_ref[0], 1.0)
    d1 = jnp.maximum(deg_ref[1], 1.0)
    acc = lax.dot_general(s_ref[0], w0_ref[...], _DN, precision=_HP) / d0
    acc = acc + lax.dot_general(s_ref[1], w1_ref[...], _DN, precision=_HP) / d1
    o_ref[...] = jnp.tanh(acc + p_ref[...])


def _tc_post(S, deg3, P, W0, W1):
    return pl.pallas_call(
        _tc_post_body,
        grid=(N // _BR,),
        in_specs=[
            pl.BlockSpec((R, _BR, D), lambda i: (0, i, 0)),
            pl.BlockSpec((R, _BR, 1), lambda i: (0, i, 0)),
            pl.BlockSpec((_BR, D), lambda i: (i, 0)),
            pl.BlockSpec((D, D), lambda i: (0, 0)),
            pl.BlockSpec((D, D), lambda i: (0, 0)),
        ],
        out_specs=pl.BlockSpec((_BR, D), lambda i: (i, 0)),
        out_shape=jax.ShapeDtypeStruct((N, D), jnp.float32),
    )(S, deg3, P, W0, W1)


def kernel(x, W, loop_weight, h_bias, edge_index_rel0, edge_index_rel1):
    src_flat = jnp.concatenate([edge_index_rel0[0], edge_index_rel1[0]])
    dst_flat = jnp.concatenate([edge_index_rel0[1], edge_index_rel1[1]])
    P = _tc_pre(x, loop_weight, h_bias.reshape(1, D))
    S, deg = _sc_segment_sum(x, src_flat, dst_flat)
    deg3 = deg.reshape(R, N_PAD, 1)
    return _tc_post(S, deg3, P, W[0], W[1])


# speedup vs baseline: 1.2659x; 1.0586x over previous
import functools

import jax
import jax.numpy as jnp
from jax import lax
from jax.experimental import pallas as pl
from jax.experimental.pallas import tpu as pltpu
from jax.experimental.pallas import tpu_sc as plsc

N = 10000
D = 128
E = 160000
R = 2
L = 16
NS = 16
N_PAD = 10240
ROWS_PER_SUB = N_PAD // NS
E_PER_SUB = E // NS
CHUNK = 80
NCHUNK = E_PER_SUB // CHUNK
NMAIN = ((NCHUNK - 2) // 3) * 3


def _sc_segment_sum(x, src0, dst0, src1, dst1):
    mesh = plsc.VectorSubcoreMesh(core_axis_name="c", subcore_axis_name="s")

    @functools.partial(
        pl.kernel,
        out_type=(jax.ShapeDtypeStruct((R, N_PAD, D), jnp.float32),
                  jax.ShapeDtypeStruct((R, N_PAD), jnp.float32)),
        mesh=mesh,
        scratch_types=[
            pltpu.VMEM((E_PER_SUB,), jnp.int32),
            pltpu.VMEM((CHUNK,), jnp.int32),
            pltpu.VMEM((CHUNK,), jnp.int32),
            pltpu.VMEM((CHUNK,), jnp.int32),
            pltpu.VMEM((CHUNK, D), jnp.float32),
            pltpu.VMEM((CHUNK, D), jnp.float32),
            pltpu.VMEM((CHUNK, D), jnp.float32),
            pltpu.VMEM((CHUNK,), jnp.float32),
            pltpu.VMEM((CHUNK,), jnp.float32),
            pltpu.VMEM_SHARED((N_PAD, D), jnp.float32),
            pltpu.VMEM_SHARED((N_PAD,), jnp.float32),
            pltpu.SemaphoreType.DMA,
            pltpu.SemaphoreType.DMA,
            pltpu.SemaphoreType.DMA,
            pltpu.SemaphoreType.DMA,
            pltpu.SemaphoreType.DMA,
            pltpu.SemaphoreType.DMA,
        ],
    )
    def sc_kernel(x_hbm, src0_hbm, dst0_hbm, src1_hbm, dst1_hbm,
                  out_hbm, deg_hbm,
                  src_v, dstb0, dstb1, dstb2, rows0, rows1, rows2,
                  ones_v, zeros_v, acc_sh, deg_sh,
                  sg0, sg1, sg2, sd0, sd1, sd2):
        c = lax.axis_index("c")
        s = lax.axis_index("s")
        ebase = s * E_PER_SUB

        one = jnp.full((L,), 1.0, jnp.float32)
        zero = jnp.zeros((L,), jnp.float32)

        @pl.loop(0, CHUNK, step=L)
        def _(i):
            ones_v[pl.ds(i, L)] = one
            zeros_v[pl.ds(i, L)] = zero

        @pl.loop(0, 64)
        def _(i):
            @pl.loop(0, D, step=L)
            def _(j):
                rows0[i, pl.ds(j, L)] = zero

        row0 = s * ROWS_PER_SUB

        @pl.loop(0, ROWS_PER_SUB, step=64)
        def _(r0):
            pltpu.sync_copy(rows0.at[pl.ds(0, 64)],
                            acc_sh.at[pl.ds(row0 + r0, 64)])
            pltpu.sync_copy(zeros_v.at[pl.ds(0, 64)],
                            deg_sh.at[pl.ds(row0 + r0, 64)])

        plsc.subcore_barrier()

        def run_relation(src_hbm, dst_hbm):
            pltpu.async_copy(
                src_hbm.at[pl.ds(ebase, E_PER_SUB)], src_v, sg0).wait()

            def gather(k, buf, sem):
                return pltpu.make_async_copy(
                    x_hbm.at[src_v.at[pl.ds(k * CHUNK, CHUNK)]], buf, sem)

            def dstcp(k, buf, sem):
                return pltpu.make_async_copy(
                    dst_hbm.at[pl.ds(ebase + k * CHUNK, CHUNK)], buf, sem)

            def consume(k, buf, dbuf, sg, sd):
                gather(k, buf, sg).wait()
                dstcp(k, dbuf, sd).wait()
                pltpu.sync_copy(buf, acc_sh.at[dbuf], add=True)
                pltpu.sync_copy(ones_v, deg_sh.at[dbuf], add=True)

            def prefetch(k, buf, dbuf, sg, sd):
                @pl.when(k < NCHUNK)
                def _():
                    dstcp(k, dbuf, sd).start()
                    gather(k, buf, sg).start()

            dstcp(0, dstb0, sd0).start()
            gather(0, rows0, sg0).start()
            dstcp(1, dstb1, sd1).start()
            gather(1, rows1, sg1).start()

            @pl.loop(0, NMAIN, step=3)
            def _(a):
                prefetch(a + 2, rows2, dstb2, sg2, sd2)
                consume(a, rows0, dstb0, sg0, sd0)
                prefetch(a + 3, rows0, dstb0, sg0, sd0)
                consume(a + 1, rows1, dstb1, sg1, sd1)
                prefetch(a + 4, rows1, dstb1, sg1, sd1)
                consume(a + 2, rows2, dstb2, sg2, sd2)

            consume(NMAIN, rows0, dstb0, sg0, sd0)
            consume(NMAIN + 1, rows1, dstb1, sg1, sd1)

        @pl.when(c == 0)
        def _():
            run_relation(src0_hbm, dst0_hbm)

        @pl.when(c == 1)
        def _():
            run_relation(src1_hbm, dst1_hbm)

        plsc.subcore_barrier()

        pltpu.sync_copy(acc_sh.at[pl.ds(row0, ROWS_PER_SUB)],
                        out_hbm.at[c, pl.ds(row0, ROWS_PER_SUB)])
        pltpu.sync_copy(deg_sh.at[pl.ds(row0, ROWS_PER_SUB)],
                        deg_hbm.at[c, pl.ds(row0, ROWS_PER_SUB)])

    return sc_kernel(x, src0, dst0, src1, dst1)


_BR = 1000
_DN = (((1,), (0,)), ((), ()))
_HP = lax.Precision.HIGHEST
_EB = 16384


def _detile_body(e0_ref, e1_ref, s0_ref, d0_ref, s1_ref, d1_ref):
    s0_ref[...] = e0_ref[0]
    d0_ref[...] = e0_ref[1]
    s1_ref[...] = e1_ref[0]
    d1_ref[...] = e1_ref[1]


def _detile(ei0, ei1):
    out1d = jax.ShapeDtypeStruct((E,), jnp.int32)
    return pl.pallas_call(
        _detile_body,
        grid=((E + _EB - 1) // _EB,),
        in_specs=[
            pl.BlockSpec((2, _EB), lambda i: (0, i)),
            pl.BlockSpec((2, _EB), lambda i: (0, i)),
        ],
        out_specs=[pl.BlockSpec((_EB,), lambda i: (i,))] * 4,
        out_shape=[out1d] * 4,
    )(ei0, ei1)


def _tc_pre_body(x_ref, lw_ref, b_ref, o_ref):
    o_ref[...] = lax.dot_general(
        x_ref[...], lw_ref[...], _DN, precision=_HP) + b_ref[...]


def _tc_pre(x, loop_w, h_bias_row):
    return pl.pallas_call(
        _tc_pre_body,
        grid=(N // _BR,),
        in_specs=[
            pl.BlockSpec((_BR, D), lambda i: (i, 0)),
            pl.BlockSpec((D, D), lambda i: (0, 0)),
            pl.BlockSpec((1, D), lambda i: (0, 0)),
        ],
        out_specs=pl.BlockSpec((_BR, D), lambda i: (i, 0)),
        out_shape=jax.ShapeDtypeStruct((N, D), jnp.float32),
    )(x, loop_w, h_bias_row)


def _tc_post_body(s_ref, deg_ref, p_ref, w0_ref, w1_ref, o_ref):
    d0 = jnp.maximum(deg_ref[0], 1.0)
    d1 = jnp.maximum(deg_ref[1], 1.0)
    acc = lax.dot_general(s_ref[0], w0_ref[...], _DN, precision=_HP) / d0
    acc = acc + lax.dot_general(s_ref[1], w1_ref[...], _DN, precision=_HP) / d1
    o_ref[...] = jnp.tanh(acc + p_ref[...])


def _tc_post(S, deg3, P, W0, W1):
    return pl.pallas_call(
        _tc_post_body,
        grid=(N // _BR,),
        in_specs=[
            pl.BlockSpec((R, _BR, D), lambda i: (0, i, 0)),
            pl.BlockSpec((R, _BR, 1), lambda i: (0, i, 0)),
            pl.BlockSpec((_BR, D), lambda i: (i, 0)),
            pl.BlockSpec((D, D), lambda i: (0, 0)),
            pl.BlockSpec((D, D), lambda i: (0, 0)),
        ],
        out_specs=pl.BlockSpec((_BR, D), lambda i: (i, 0)),
        out_shape=jax.ShapeDtypeStruct((N, D), jnp.float32),
    )(S, deg3, P, W0, W1)


def kernel(x, W, loop_weight, h_bias, edge_index_rel0, edge_index_rel1):
    src0, dst0, src1, dst1 = _detile(edge_index_rel0, edge_index_rel1)
    P = _tc_pre(x, loop_weight, h_bias.reshape(1, D))
    S, deg = _sc_segment_sum(x, src0, dst0, src1, dst1)
    deg3 = deg.reshape(R, N_PAD, 1)
    return _tc_post(S, deg3, P, W[0], W[1])
